# baseline (speedup 1.0000x reference)
import jax
import jax.numpy as jnp
from jax import lax
from jax.experimental import pallas as pl
from jax.experimental.pallas import tpu as pltpu
from jax.experimental.pallas import tpu_sc as plsc

_L = 17
_N = 384
_PLANES = 6
_NW = 32
_RPW = _N // _NW
_WIN = _RPW + 4
_TAB = 4920
_GROUPS = _N // 16
_OW = 2 * _N

_OFFS = {'h': ((0, 1), (0, 2)), 'd': ((1, 1), (2, 2)),
         'b': ((1, 2), (2, 1)), 'v': ((1, 0), (2, 0))}


def _rot_disp(dy, dx, r):
    return [(dy, dx), (dx, -dy), (-dy, -dx), (-dx, dy)][r]


def _out_perm(u, v, r):
    return [(u, v), (v, 1 - u), (1 - u, 1 - v), (1 - v, u)][r]


def _idx_groups():
    groups = {}
    order = []
    for ki, k in enumerate(['h', 'd', 'b', 'v']):
        (o1, o2) = _OFFS[k]
        for r in range(4):
            d1 = _rot_disp(o1[0], o1[1], r)
            d2 = _rot_disp(o2[0], o2[1], r)
            perm = [0] * 4
            for u in (0, 1):
                for v in (0, 1):
                    up, vp = _out_perm(u, v, r)
                    perm[u * 2 + v] = up * 2 + vp
            key = (d1, d2)
            if key not in groups:
                groups[key] = []
                order.append(key)
            groups[key].append((ki, tuple(perm)))
    return [(d1, d2, groups[(d1, d2)]) for (d1, d2) in order]


_IDX_GROUPS = _idx_groups()
_DYS = (-2, -1, 0, 1, 2)

_TAB_PLAN = []
_SHARED_COL = {}
_NUM_PAIRS = 0
for (_d1, _d2, _members) in _IDX_GROUPS:
    if len(_members) > 1:
        _TAB_PLAN.append((_NUM_PAIRS, None))
        _NUM_PAIRS += 1
    else:
        (_ki, _perm) = _members[0]
        if _ki not in _SHARED_COL:
            _SHARED_COL[_ki] = _NUM_PAIRS
            _NUM_PAIRS += 1
        _TAB_PLAN.append((_SHARED_COL[_ki], _perm))


def _body(img_ref, tabs_ref, out_ref, *scratch):
    tab_refs = scratch[0:2 * _NUM_PAIRS]
    inbufs = scratch[2 * _NUM_PAIRS:2 * _NUM_PAIRS + 2]
    outbufs = scratch[2 * _NUM_PAIRS + 2:2 * _NUM_PAIRS + 4]
    sem_tab = scratch[2 * _NUM_PAIRS + 4]
    sem_in = scratch[2 * _NUM_PAIRS + 5:2 * _NUM_PAIRS + 7]
    sem_out = scratch[2 * _NUM_PAIRS + 7:2 * _NUM_PAIRS + 9]

    cid = lax.axis_index("c")
    sid = lax.axis_index("s")
    wid = sid * 2 + cid
    row0 = wid * _RPW
    ws = jnp.maximum(jnp.minimum(row0 - 2, _N - _WIN), 0)

    tab_copies = [
        pltpu.async_copy(tabs_ref.at[pl.ds(i * _TAB, _TAB)], tab_refs[i],
                         sem_tab)
        for i in range(2 * _NUM_PAIRS)]

    def in_copy(t):
        return pltpu.async_copy(
            img_ref.at[pl.ds(t * _N * _N + ws * _N, _WIN * _N)],
            inbufs[t % 2], sem_in[t % 2])

    in_handles = {0: in_copy(0)}
    out_handles = {}

    for c in tab_copies:
        c.wait()

    iota = lax.iota(jnp.int32, 16)
    iota2 = iota * 2

    for t in range(_PLANES):
        in_handles[t].wait()
        if t + 1 < _PLANES:
            in_handles[t + 1] = in_copy(t + 1)
        if t >= 2:
            out_handles[t - 2].wait()
        inbuf = inbufs[t % 2]
        outbuf = outbufs[t % 2]

        def pair_body(p, carry):
            y0 = row0 + 2 * p
            rbs = []
            for j in range(6):
                yy = jnp.maximum(jnp.minimum(y0 - 2 + j, _N - 1), 0)
                rbs.append((yy - ws) * _N)

            def grp_body(g, c2):
              for gu in range(2):
                x = (g * 2 + gu) * 16
                cvs = {}
                for dx in _DYS:
                    cvs[dx] = jnp.maximum(
                        jnp.minimum(iota + (x + dx), _N - 1), 0)
                val = {}
                for j in range(6):
                    for dx in _DYS:
                        val[(j, dx)] = plsc.load_gather(
                            inbuf, [cvs[dx] + rbs[j]])
                for r in (0, 1):
                    nb = {(dy, dx): val[(dy + 2 + r, dx)]
                          for dy in _DYS for dx in _DYS}
                    a289 = nb[(0, 0)] * (_L * _L)
                    accs = [None] * 4
                    for gi, (d1, d2, members) in enumerate(_IDX_GROUPS):
                        (pair_i, perm) = _TAB_PLAN[gi]
                        if perm is None:
                            perm = (0, 1, 2, 3)
                        idx = a289 + nb[d1] * _L + nb[d2]
                        pk_t = plsc.load_gather(tab_refs[2 * pair_i],
                                                [idx])
                        pk_b = plsc.load_gather(tab_refs[2 * pair_i + 1],
                                                [idx])
                        w = (plsc.bitcast(lax.shift_left(pk_t, 16),
                                          jnp.float32),
                             plsc.bitcast(pk_t, jnp.float32),
                             plsc.bitcast(lax.shift_left(pk_b, 16),
                                          jnp.float32),
                             plsc.bitcast(pk_b, jnp.float32))
                        for uv in range(4):
                            s = perm[uv]
                            accs[s] = (w[uv] if accs[s] is None
                                       else accs[s] + w[uv])
                    lr = 2 * (2 * p + r)
                    rv0 = jnp.full((16,), lr, jnp.int32)
                    rv1 = jnp.full((16,), lr + 1, jnp.int32)
                    cv0 = iota2 + 2 * x
                    cv1 = cv0 + 1
                    plsc.store_scatter(outbuf, [rv0, cv0], accs[0])
                    plsc.store_scatter(outbuf, [rv0, cv1], accs[1])
                    plsc.store_scatter(outbuf, [rv1, cv0], accs[2])
                    plsc.store_scatter(outbuf, [rv1, cv1], accs[3])
              return c2

            lax.fori_loop(0, _GROUPS // 2, grp_body, 0)
            return carry

        lax.fori_loop(0, _RPW // 2, pair_body, 0)
        out_handles[t] = pltpu.async_copy(
            outbuf,
            out_ref.at[t // 3, t % 3, pl.ds(2 * row0, 2 * _RPW), :],
            sem_out[t % 2])

    out_handles[_PLANES - 2].wait()
    out_handles[_PLANES - 1].wait()


def _sel_matrix():
    import numpy as np
    S = np.zeros((2 * _NUM_PAIRS * 2, 16), np.float32)
    done = set()
    for gi, (d1, d2, members) in enumerate(_IDX_GROUPS):
        (pair_i, _) = _TAB_PLAN[gi]
        if pair_i in done:
            continue
        done.add(pair_i)
        if len(members) > 1:
            for s in range(4):
                row = (2 * pair_i + s // 2) * 2 + (s % 2)
                for (ki, perm) in members:
                    S[row, ki * 4 + perm.index(s)] = 0.25
        else:
            (ki, _) = members[0]
            for uv in range(4):
                row = (2 * pair_i + uv // 2) * 2 + (uv % 2)
                S[row, ki * 4 + uv] = 0.25
    return S


_SEL = _sel_matrix()


def kernel(img_lr, h_weight, d_weight, b_weight, v_weight):
    img = img_lr.astype(jnp.int32).reshape(_PLANES * _N * _N)

    Wcat = jnp.concatenate(
        [w.reshape(_L ** 3, 4)
         for w in (h_weight, d_weight, b_weight, v_weight)],
        axis=1)
    C = Wcat @ jnp.asarray(_SEL).T
    bits = lax.bitcast_convert_type(
        C.astype(jnp.bfloat16), jnp.uint16).astype(jnp.uint32)
    words = lax.bitcast_convert_type(
        (bits[:, 1::2] << 16) | bits[:, 0::2], jnp.int32)
    tabs = jnp.pad(words.T, ((0, 0), (0, _TAB - _L ** 3))).reshape(-1)

    mesh = plsc.VectorSubcoreMesh(core_axis_name="c", subcore_axis_name="s")
    scratch = [pltpu.VMEM((_TAB,), jnp.int32)
               for _ in range(2 * _NUM_PAIRS)]
    scratch += [pltpu.VMEM((_WIN * _N,), jnp.int32) for _ in range(2)]
    scratch += [pltpu.VMEM((2 * _RPW, _OW), jnp.float32)
                for _ in range(2)]
    scratch += [pltpu.SemaphoreType.DMA for _ in range(5)]

    out = pl.kernel(
        _body,
        out_type=jax.ShapeDtypeStruct((2, 3, _OW, _OW), jnp.float32),
        mesh=mesh,
        scratch_types=scratch,
        compiler_params=pltpu.CompilerParams(needs_layout_passes=False,
                                             use_tc_tiling_on_sc=True),
    )(img, tabs)
    return out


# speedup vs baseline: 1.0157x; 1.0157x over previous
import jax
import jax.numpy as jnp
from jax import lax
from jax.experimental import pallas as pl
from jax.experimental.pallas import tpu as pltpu
from jax.experimental.pallas import tpu_sc as plsc

_L = 17
_N = 384
_PLANES = 6
_NW = 32
_RPW = _N // _NW
_WIN = _RPW + 4
_TAB = 4920
_GROUPS = _N // 16
_OW = 2 * _N

_OFFS = {'h': ((0, 1), (0, 2)), 'd': ((1, 1), (2, 2)),
         'b': ((1, 2), (2, 1)), 'v': ((1, 0), (2, 0))}


def _rot_disp(dy, dx, r):
    return [(dy, dx), (dx, -dy), (-dy, -dx), (-dx, dy)][r]


def _out_perm(u, v, r):
    return [(u, v), (v, 1 - u), (1 - u, 1 - v), (1 - v, u)][r]


def _idx_groups():
    groups = {}
    order = []
    for ki, k in enumerate(['h', 'd', 'b', 'v']):
        (o1, o2) = _OFFS[k]
        for r in range(4):
            d1 = _rot_disp(o1[0], o1[1], r)
            d2 = _rot_disp(o2[0], o2[1], r)
            perm = [0] * 4
            for u in (0, 1):
                for v in (0, 1):
                    up, vp = _out_perm(u, v, r)
                    perm[u * 2 + v] = up * 2 + vp
            key = (d1, d2)
            if key not in groups:
                groups[key] = []
                order.append(key)
            groups[key].append((ki, tuple(perm)))
    return [(d1, d2, groups[(d1, d2)]) for (d1, d2) in order]


_IDX_GROUPS = _idx_groups()
_DYS = (-2, -1, 0, 1, 2)

_TAB_PLAN = []
_SHARED_COL = {}
_NUM_PAIRS = 0
for (_d1, _d2, _members) in _IDX_GROUPS:
    if len(_members) > 1:
        _TAB_PLAN.append((_NUM_PAIRS, None))
        _NUM_PAIRS += 1
    else:
        (_ki, _perm) = _members[0]
        if _ki not in _SHARED_COL:
            _SHARED_COL[_ki] = _NUM_PAIRS
            _NUM_PAIRS += 1
        _TAB_PLAN.append((_SHARED_COL[_ki], _perm))


def _body(img_ref, tabs_ref, out_ref, *scratch):
    tab_refs = scratch[0:2 * _NUM_PAIRS]
    inbufs = scratch[2 * _NUM_PAIRS:2 * _NUM_PAIRS + 2]
    outbufs = scratch[2 * _NUM_PAIRS + 2:2 * _NUM_PAIRS + 4]
    sem_tab = scratch[2 * _NUM_PAIRS + 4]
    sem_in = scratch[2 * _NUM_PAIRS + 5:2 * _NUM_PAIRS + 7]
    sem_out = scratch[2 * _NUM_PAIRS + 7:2 * _NUM_PAIRS + 9]

    cid = lax.axis_index("c")
    sid = lax.axis_index("s")
    wid = sid * 2 + cid
    row0 = wid * _RPW
    ws = jnp.maximum(jnp.minimum(row0 - 2, _N - _WIN), 0)

    tab_copies = [
        pltpu.async_copy(tabs_ref.at[pl.ds(i * _TAB, _TAB)], tab_refs[i],
                         sem_tab)
        for i in range(2 * _NUM_PAIRS)]

    def in_copy(t):
        return pltpu.async_copy(
            img_ref.at[pl.ds(t * _N * _N + ws * _N, _WIN * _N)],
            inbufs[t % 2], sem_in[t % 2])

    in_handles = {0: in_copy(0)}
    out_handles = {}

    for c in tab_copies:
        c.wait()

    iota = lax.iota(jnp.int32, 16)
    iota2 = iota * 2

    for t in range(_PLANES):
        in_handles[t].wait()
        if t + 1 < _PLANES:
            in_handles[t + 1] = in_copy(t + 1)
        if t >= 2:
            out_handles[t - 2].wait()
        inbuf = inbufs[t % 2]
        outbuf = outbufs[t % 2]

        def pair_body(p, carry):
            y0 = row0 + 2 * p
            rbs = []
            for j in range(6):
                yy = jnp.maximum(jnp.minimum(y0 - 2 + j, _N - 1), 0)
                rbs.append((yy - ws) * _N)

            def grp_body(g, c2):
              for gu in range(2):
                x = (g * 2 + gu) * 16
                cvs = {}
                for dx in _DYS:
                    cv = iota + (x + dx)
                    if dx < 0:
                        cv = jnp.maximum(cv, 0)
                    elif dx > 0:
                        cv = jnp.minimum(cv, _N - 1)
                    cvs[dx] = cv
                val = {}
                for j in range(6):
                    for dx in _DYS:
                        val[(j, dx)] = plsc.load_gather(
                            inbuf, [cvs[dx] + rbs[j]])
                for r in (0, 1):
                    nb = {(dy, dx): val[(dy + 2 + r, dx)]
                          for dy in _DYS for dx in _DYS}
                    a289 = nb[(0, 0)] * (_L * _L)
                    accs = [None] * 4
                    for gi, (d1, d2, members) in enumerate(_IDX_GROUPS):
                        (pair_i, perm) = _TAB_PLAN[gi]
                        if perm is None:
                            perm = (0, 1, 2, 3)
                        idx = a289 + nb[d1] * _L + nb[d2]
                        pk_t = plsc.load_gather(tab_refs[2 * pair_i],
                                                [idx])
                        pk_b = plsc.load_gather(tab_refs[2 * pair_i + 1],
                                                [idx])
                        w = (plsc.bitcast(lax.shift_left(pk_t, 16),
                                          jnp.float32),
                             plsc.bitcast(pk_t, jnp.float32),
                             plsc.bitcast(lax.shift_left(pk_b, 16),
                                          jnp.float32),
                             plsc.bitcast(pk_b, jnp.float32))
                        for uv in range(4):
                            s = perm[uv]
                            accs[s] = (w[uv] if accs[s] is None
                                       else accs[s] + w[uv])
                    lr = 2 * (2 * p + r)
                    rv0 = jnp.full((16,), lr, jnp.int32)
                    rv1 = jnp.full((16,), lr + 1, jnp.int32)
                    cv0 = iota2 + 2 * x
                    cv1 = cv0 + 1
                    plsc.store_scatter(outbuf, [rv0, cv0], accs[0])
                    plsc.store_scatter(outbuf, [rv0, cv1], accs[1])
                    plsc.store_scatter(outbuf, [rv1, cv0], accs[2])
                    plsc.store_scatter(outbuf, [rv1, cv1], accs[3])
              return c2

            lax.fori_loop(0, _GROUPS // 2, grp_body, 0)
            return carry

        lax.fori_loop(0, _RPW // 2, pair_body, 0)
        out_handles[t] = pltpu.async_copy(
            outbuf,
            out_ref.at[t // 3, t % 3, pl.ds(2 * row0, 2 * _RPW), :],
            sem_out[t % 2])

    out_handles[_PLANES - 2].wait()
    out_handles[_PLANES - 1].wait()


def _sel_matrix():
    import numpy as np
    S = np.zeros((2 * _NUM_PAIRS * 2, 16), np.float32)
    done = set()
    for gi, (d1, d2, members) in enumerate(_IDX_GROUPS):
        (pair_i, _) = _TAB_PLAN[gi]
        if pair_i in done:
            continue
        done.add(pair_i)
        if len(members) > 1:
            for s in range(4):
                row = (2 * pair_i + s // 2) * 2 + (s % 2)
                for (ki, perm) in members:
                    S[row, ki * 4 + perm.index(s)] = 0.25
        else:
            (ki, _) = members[0]
            for uv in range(4):
                row = (2 * pair_i + uv // 2) * 2 + (uv % 2)
                S[row, ki * 4 + uv] = 0.25
    return S


_SEL = _sel_matrix()


def kernel(img_lr, h_weight, d_weight, b_weight, v_weight):
    img = img_lr.astype(jnp.int32).reshape(_PLANES * _N * _N)

    W = jnp.concatenate(
        [w.reshape(_L ** 3, 4).T
         for w in (h_weight, d_weight, b_weight, v_weight)])
    C = jnp.asarray(_SEL) @ W
    bits = lax.bitcast_convert_type(
        C.astype(jnp.bfloat16), jnp.uint16).astype(jnp.uint32)
    bits = bits.reshape(2 * _NUM_PAIRS, 2, _L ** 3)
    words = lax.bitcast_convert_type(
        (bits[:, 1, :] << 16) | bits[:, 0, :], jnp.int32)
    tabs = jnp.pad(words, ((0, 0), (0, _TAB - _L ** 3))).reshape(-1)

    mesh = plsc.VectorSubcoreMesh(core_axis_name="c", subcore_axis_name="s")
    scratch = [pltpu.VMEM((_TAB,), jnp.int32)
               for _ in range(2 * _NUM_PAIRS)]
    scratch += [pltpu.VMEM((_WIN * _N,), jnp.int32) for _ in range(2)]
    scratch += [pltpu.VMEM((2 * _RPW, _OW), jnp.float32)
                for _ in range(2)]
    scratch += [pltpu.SemaphoreType.DMA for _ in range(5)]

    out = pl.kernel(
        _body,
        out_type=jax.ShapeDtypeStruct((2, 3, _OW, _OW), jnp.float32),
        mesh=mesh,
        scratch_types=scratch,
        compiler_params=pltpu.CompilerParams(needs_layout_passes=False,
                                             use_tc_tiling_on_sc=True),
    )(img, tabs)
    return out
